# baseline (speedup 1.0000x reference)
import functools

import jax
import jax.numpy as jnp
from jax import lax
from jax.experimental import pallas as pl
from jax.experimental.pallas import tpu as pltpu
from jax.experimental.pallas import tpu_sc as plsc

VOCAB = 1000000
DIM = 64
B = 16384
CTX = 10
NEG = 20
NT = 1 + NEG

NC = 2
NS = 16
NW = NC * NS
L = 16
DCH = DIM // L

HROWS = VOCAB // 2
PD = 2 * DIM

EPW = B // NW
CE = 16
NCHUNK = EPW // CE
CTX_I = CE * CTX
OUT_I = CE * NT

W = 256
NBLK = VOCAB // W
REM = VOCAB - NBLK * W
PITCH = W + 1

LN2 = 0.6931471805599453
_LOG2P = (5.0603279536654e-06, 1.4423955889439504, -0.7169875678728092,
          0.4538582052898957, -0.272355827037999, 0.11790686114989256,
          -0.024825984442586733)


def _ln(v):
    bits = plsc.bitcast(v, jnp.int32)
    e = (bits >> 23) - 127
    m = plsc.bitcast((bits & 0x007FFFFF) | 0x3F800000, jnp.float32)
    t = m - 1.0
    p = jnp.full((L,), _LOG2P[-1], dtype=jnp.float32)
    for c in reversed(_LOG2P[:-1]):
        p = p * t + c
    return (e.astype(jnp.float32) + p) * LN2


def _relayout_body(in_t, out_t, in_rm, out_rm, blk_v, pk_v, sem):
    wid = lax.axis_index("s") * NC + lax.axis_index("c")
    diota = lax.iota(jnp.int32, L)

    def do_block(src, dst, v0, width):
        v0 = pl.multiple_of(v0, 128)
        rd = max(width, 128)
        pltpu.async_copy(src.at[:, pl.ds(v0, rd)],
                         blk_v.at[:, pl.ds(0, rd)], sem).wait()

        @pl.loop(0, width // 2, unroll=8)
        def _col(h):
            for par in range(2):
                dv = h * 2 + par
                for j in range(DCH):
                    col = plsc.load_gather(
                        blk_v, [diota + j * L,
                                jnp.full((L,), 1, jnp.int32) * dv])
                    pk_v[h, pl.ds(par * DIM + j * L, L)] = col

        pltpu.sync_copy(pk_v.at[pl.ds(0, width // 2)],
                        dst.at[pl.ds(pl.multiple_of(v0 // 2, 64),
                                     width // 2)])

    for src, dst in ((in_t, in_rm), (out_t, out_rm)):
        nper = NBLK // NW

        @pl.loop(0, nper)
        def _blk(k):
            do_block(src, dst, (k * NW + wid) * W, W)

        extra = NBLK - nper * NW
        for e in range(extra):
            @pl.when(wid == NW - 1 - e)
            def _():
                do_block(src, dst, (nper * NW + e) * W, W)

        @pl.when(wid == NW - 1 - extra)
        def _():
            do_block(src, dst, NBLK * W, REM)


_relayout = functools.partial(
    pl.kernel,
    out_type=(jax.ShapeDtypeStruct((HROWS, PD), jnp.float32),
              jax.ShapeDtypeStruct((HROWS, PD), jnp.float32)),
    mesh=plsc.VectorSubcoreMesh(core_axis_name="c", subcore_axis_name="s"),
    scratch_types=[
        pltpu.VMEM((DIM, PITCH), jnp.float32),
        pltpu.VMEM((W // 2, PD), jnp.float32),
        pltpu.SemaphoreType.DMA,
    ],
    compiler_params=pltpu.CompilerParams(use_tc_tiling_on_sc=True,
                                         needs_layout_passes=False),
)(_relayout_body)


def _sc_body(ctx_hbm, oidx_hbm, in_rm, out_rm, loss_hbm,
             ctx_idx_v, out_idx_v, ctx_sh_v, out_sh_v,
             ctx_rows_v, out_rows_v, scores_v, acc_v, sem):
    wid = lax.axis_index("s") * NC + lax.axis_index("c")
    base = wid * EPW
    acc_v[...] = jnp.zeros((L,), jnp.float32)

    @pl.loop(0, NCHUNK)
    def _chunk(c):
        e0 = base + c * CE

        pltpu.sync_copy(ctx_hbm.at[pl.ds(e0 * CTX, CTX_I)],
                        ctx_idx_v.at[pl.ds(0, CTX_I)])
        pltpu.sync_copy(oidx_hbm.at[pl.ds(e0 * NT, OUT_I)],
                        out_idx_v.at[pl.ds(0, OUT_I)])
        for g in range(CTX_I // L):
            ctx_sh_v[pl.ds(g * L, L)] = ctx_idx_v[pl.ds(g * L, L)] >> 1
        for g in range(OUT_I // L):
            out_sh_v[pl.ds(g * L, L)] = out_idx_v[pl.ds(g * L, L)] >> 1

        cps = []
        for g in range(2):
            cps.append(pltpu.async_copy(
                in_rm.at[ctx_sh_v.at[pl.ds(g * 80, 80)]],
                ctx_rows_v.at[pl.ds(g * 80, 80)], sem))
        for g in range(3):
            cps.append(pltpu.async_copy(
                out_rm.at[out_sh_v.at[pl.ds(g * 112, 112)]],
                out_rows_v.at[pl.ds(g * 112, 112)], sem))
        for cp in cps:
            cp.wait()

        @pl.loop(0, CE)
        def _elem(i):
            cbase = i * CTX
            cpar = (ctx_idx_v[pl.ds(cbase, L)] & 1) * DIM
            cm = []
            for j in range(DCH):
                a = ctx_rows_v[cbase, pl.ds(cpar[0] + j * L, L)]
                for r in range(1, CTX):
                    a = a + ctx_rows_v[cbase + r, pl.ds(cpar[r] + j * L, L)]
                cm.append(a * (1.0 / CTX))
            obase = i * NT
            opar1 = (out_idx_v[pl.ds(obase, L)] & 1) * DIM
            opar2 = (out_idx_v[pl.ds(obase + NT - L, L)] & 1) * DIM
            for t in range(NT):
                ot = opar1[t] if t < L else opar2[t - (NT - L)]
                p = cm[0] * out_rows_v[obase + t, pl.ds(ot, L)]
                for j in range(1, DCH):
                    p = p + cm[j] * out_rows_v[obase + t, pl.ds(ot + j * L, L)]
                scores_v[obase + t, pl.ds(0, L)] = p

        riota = lax.iota(jnp.int32, L)
        tot = None
        for t in range(NT):
            rows = riota * NT + t
            s = plsc.load_gather(scores_v, [rows, jnp.zeros((L,), jnp.int32)])
            for l in range(1, L):
                s = s + plsc.load_gather(
                    scores_v, [rows, jnp.full((L,), l, jnp.int32)])
            sg = 1.0 / (1.0 + jnp.exp(-s))
            if t == 0:
                tot = _ln(sg + 1e-10)
            else:
                tot = tot + _ln((1.0 - sg) + 1e-10)
        acc_v[...] += tot

    pltpu.sync_copy(acc_v, loss_hbm.at[wid])


_sc_loss = functools.partial(
    pl.kernel,
    out_type=jax.ShapeDtypeStruct((NW, L), jnp.float32),
    mesh=plsc.VectorSubcoreMesh(core_axis_name="c", subcore_axis_name="s"),
    scratch_types=[
        pltpu.VMEM((CTX_I + L,), jnp.int32),
        pltpu.VMEM((OUT_I + L,), jnp.int32),
        pltpu.VMEM((CTX_I,), jnp.int32),
        pltpu.VMEM((OUT_I,), jnp.int32),
        pltpu.VMEM((CTX_I, PD), jnp.float32),
        pltpu.VMEM((OUT_I, PD), jnp.float32),
        pltpu.VMEM((OUT_I, L + 1), jnp.float32),
        pltpu.VMEM((L,), jnp.float32),
        pltpu.SemaphoreType.DMA,
    ],
    compiler_params=pltpu.CompilerParams(use_tc_tiling_on_sc=True,
                                         needs_layout_passes=False),
)(_sc_body)


def _tc_body(part_ref, o_ref):
    o_ref[0, 0] = -jnp.sum(part_ref[...]) * (1.0 / B)


_tc_sum = pl.pallas_call(
    _tc_body,
    out_specs=pl.BlockSpec(memory_space=pltpu.SMEM),
    out_shape=jax.ShapeDtypeStruct((1, 1), jnp.float32),
)


@jax.jit
def kernel(context_idxs, target_idx, negative_idxs, in_embed, out_embed):
    ctx_flat = context_idxs.astype(jnp.int32).reshape(B * CTX)
    out_idx = jnp.concatenate(
        [target_idx.astype(jnp.int32)[:, None],
         negative_idxs.astype(jnp.int32)], axis=1).reshape(B * NT)
    in_rm = in_embed.reshape(HROWS, PD)
    out_rm = out_embed.reshape(HROWS, PD)
    part = _sc_loss(ctx_flat, out_idx, in_rm, out_rm)
    return _tc_sum(part)[0, 0]


# speedup vs baseline: 1.0788x; 1.0788x over previous
import functools

import jax
import jax.numpy as jnp
from jax import lax
from jax.experimental import pallas as pl
from jax.experimental.pallas import tpu as pltpu
from jax.experimental.pallas import tpu_sc as plsc

VOCAB = 1000000
DIM = 64
B = 16384
CTX = 10
NEG = 20
NT = 1 + NEG

NC = 2
NS = 16
NW = NC * NS
L = 16
DCH = DIM // L

EPW = B // NW
CE = 16
NCHUNK = EPW // CE
CTX_I = CE * CTX
OUT_I = CE * NT

LN2 = 0.6931471805599453
_LOG2P = (5.0603279536654e-06, 1.4423955889439504, -0.7169875678728092,
          0.4538582052898957, -0.272355827037999, 0.11790686114989256,
          -0.024825984442586733)


def _ln(v):
    bits = plsc.bitcast(v, jnp.int32)
    e = (bits >> 23) - 127
    m = plsc.bitcast((bits & 0x007FFFFF) | 0x3F800000, jnp.float32)
    t = m - 1.0
    p = jnp.full((L,), _LOG2P[-1], dtype=jnp.float32)
    for c in reversed(_LOG2P[:-1]):
        p = p * t + c
    return (e.astype(jnp.float32) + p) * LN2


def _sc_body(ctx_hbm, oidx_hbm, in_emb, out_emb, loss_hbm,
             ctx_idx_v, out_idx_v, ctx_rows_v, out_rows_v, scores_v, acc_v,
             sem):
    wid = lax.axis_index("s") * NC + lax.axis_index("c")
    base = wid * EPW
    acc_v[...] = jnp.zeros((L,), jnp.float32)

    @pl.loop(0, NCHUNK)
    def _chunk(c):
        e0 = base + c * CE

        pltpu.sync_copy(ctx_hbm.at[pl.ds(e0 * CTX, CTX_I)], ctx_idx_v)
        pltpu.sync_copy(oidx_hbm.at[pl.ds(e0 * NT, OUT_I)], out_idx_v)

        cps = []
        for g in range(2):
            cps.append(pltpu.async_copy(
                in_emb.at[ctx_idx_v.at[pl.ds(g * 80, 80)]],
                ctx_rows_v.at[pl.ds(g * 80, 80)], sem))
        for g in range(3):
            cps.append(pltpu.async_copy(
                out_emb.at[out_idx_v.at[pl.ds(g * 112, 112)]],
                out_rows_v.at[pl.ds(g * 112, 112)], sem))
        for cp in cps:
            cp.wait()

        @pl.loop(0, CE)
        def _elem(i):
            cbase = i * CTX
            cm = []
            for j in range(DCH):
                a = ctx_rows_v[cbase, pl.ds(j * L, L)]
                for r in range(1, CTX):
                    a = a + ctx_rows_v[cbase + r, pl.ds(j * L, L)]
                cm.append(a * (1.0 / CTX))
            obase = i * NT
            for t in range(NT):
                p = cm[0] * out_rows_v[obase + t, pl.ds(0, L)]
                for j in range(1, DCH):
                    p = p + cm[j] * out_rows_v[obase + t, pl.ds(j * L, L)]
                scores_v[obase + t, pl.ds(0, L)] = p

        riota = lax.iota(jnp.int32, L)
        tot = None
        for t in range(NT):
            rows = riota * NT + t
            s = plsc.load_gather(scores_v, [rows, jnp.zeros((L,), jnp.int32)])
            for l in range(1, L):
                s = s + plsc.load_gather(
                    scores_v, [rows, jnp.full((L,), l, jnp.int32)])
            sg = 1.0 / (1.0 + jnp.exp(-s))
            if t == 0:
                tot = _ln(sg + 1e-10)
            else:
                tot = tot + _ln((1.0 - sg) + 1e-10)
        acc_v[...] += tot

    pltpu.sync_copy(acc_v, loss_hbm.at[wid])


_sc_loss = functools.partial(
    pl.kernel,
    out_type=jax.ShapeDtypeStruct((NW, L), jnp.float32),
    mesh=plsc.VectorSubcoreMesh(core_axis_name="c", subcore_axis_name="s"),
    scratch_types=[
        pltpu.VMEM((CTX_I,), jnp.int32),
        pltpu.VMEM((OUT_I,), jnp.int32),
        pltpu.VMEM((CTX_I, DIM), jnp.float32),
        pltpu.VMEM((OUT_I, DIM), jnp.float32),
        pltpu.VMEM((OUT_I, L + 1), jnp.float32),
        pltpu.VMEM((L,), jnp.float32),
        pltpu.SemaphoreType.DMA,
    ],
    compiler_params=pltpu.CompilerParams(use_tc_tiling_on_sc=False,
                                         needs_layout_passes=False),
)(_sc_body)


def _tc_body(part_ref, o_ref):
    o_ref[0, 0] = -jnp.sum(part_ref[...]) * (1.0 / B)


_tc_sum = pl.pallas_call(
    _tc_body,
    out_specs=pl.BlockSpec(memory_space=pltpu.SMEM),
    out_shape=jax.ShapeDtypeStruct((1, 1), jnp.float32),
)


@jax.jit
def kernel(context_idxs, target_idx, negative_idxs, in_embed, out_embed):
    ctx_flat = context_idxs.astype(jnp.int32).reshape(B * CTX)
    out_idx = jnp.concatenate(
        [target_idx.astype(jnp.int32)[:, None],
         negative_idxs.astype(jnp.int32)], axis=1).reshape(B * NT)
    part = _sc_loss(ctx_flat, out_idx, in_embed, out_embed)
    return _tc_sum(part)[0, 0]
